# Initial kernel scaffold; baseline (speedup 1.0000x reference)
import jax
import jax.numpy as jnp
from jax.experimental import pallas as pl


def kernel(x, params, pos_enc):
    raise NotImplementedError("write your pallas kernel here")


# speedup vs baseline: 1.1794x; 1.1794x over previous
import functools
import math

import jax
import jax.numpy as jnp
from jax import lax
from jax.experimental import pallas as pl
from jax.experimental.pallas import tpu as pltpu

B, S, IN, D, F, E, K, L, H, OUT = 2, 512, 128, 1024, 4096, 8, 2, 2, 16, 3
N = B * S
DH = D // H
NT = N // 128

_bf = jnp.bfloat16


def _dotT(a, b):
    return lax.dot_general(a, b, (((1,), (1,)), ((), ())),
                           preferred_element_type=jnp.float32)


def _split_bf(a):
    hi = a.astype(_bf)
    lo = (a - hi.astype(jnp.float32)).astype(_bf)
    return hi, lo


def _dot3(a, b, dims):
    ah, al = _split_bf(a)
    bh, bl = _split_bf(b)
    dn = (dims, ((), ()))
    r = lax.dot_general(ah, bh, dn, preferred_element_type=jnp.float32)
    r += lax.dot_general(al, bh, dn, preferred_element_type=jnp.float32)
    r += lax.dot_general(ah, bl, dn, preferred_element_type=jnp.float32)
    return r


def _dotT32(a, b):
    return _dot3(a, b, ((1,), (1,)))


def _gelu(x):
    return 0.5 * x * (1.0 + lax.erf(x * (1.0 / math.sqrt(2.0))))


def _ln(y, g, b):
    m = jnp.mean(y, axis=-1, keepdims=True)
    v = jnp.mean((y - m) ** 2, axis=-1, keepdims=True)
    return (y - m) * lax.rsqrt(v + 1e-5) * g + b


def _inproj_body(x_ref, w_ref, b_ref, pe_ref, o_ref, obf_ref):
    y = _dotT32(x_ref[...], w_ref[...]) + b_ref[...] + pe_ref[...]
    o_ref[...] = y
    obf_ref[...] = y.astype(_bf)


def _inproj(x2d, w_bf, bias, pe):
    return pl.pallas_call(
        _inproj_body,
        grid=(NT,),
        in_specs=[
            pl.BlockSpec((128, IN), lambda t: (t, 0)),
            pl.BlockSpec((D, IN), lambda t: (0, 0)),
            pl.BlockSpec((1, D), lambda t: (0, 0)),
            pl.BlockSpec((128, D), lambda t: (t % (S // 128), 0)),
        ],
        out_specs=[
            pl.BlockSpec((128, D), lambda t: (t, 0)),
            pl.BlockSpec((128, D), lambda t: (t, 0)),
        ],
        out_shape=[
            jax.ShapeDtypeStruct((N, D), jnp.float32),
            jax.ShapeDtypeStruct((N, D), _bf),
        ],
    )(x2d, w_bf, bias, pe)


def _attn_body(h_ref, wq_ref, wk_ref, wv_ref, bq_ref, bk_ref, bv_ref, o_ref):
    hb = h_ref[0]
    q = _dotT(hb, wq_ref[0]) + bq_ref[0]
    k = _dotT(hb, wk_ref[0]) + bk_ref[0]
    v = _dotT(hb, wv_ref[0]) + bv_ref[0]
    s = _dotT32(q, k) * (1.0 / math.sqrt(DH))
    m = jnp.max(s, axis=-1, keepdims=True)
    p = jnp.exp(s - m)
    p = p / jnp.sum(p, axis=-1, keepdims=True)
    o = _dot3(p, v, ((1,), (0,)))
    o_ref[0, 0] = o


def _attention(h_bf, wq, wk, wv, bq, bk, bv):
    out = pl.pallas_call(
        _attn_body,
        grid=(B, H),
        in_specs=[
            pl.BlockSpec((1, S, D), lambda b, h: (b, 0, 0)),
            pl.BlockSpec((1, DH, D), lambda b, h: (h, 0, 0)),
            pl.BlockSpec((1, DH, D), lambda b, h: (h, 0, 0)),
            pl.BlockSpec((1, DH, D), lambda b, h: (h, 0, 0)),
            pl.BlockSpec((1, 1, DH), lambda b, h: (h, 0, 0)),
            pl.BlockSpec((1, 1, DH), lambda b, h: (h, 0, 0)),
            pl.BlockSpec((1, 1, DH), lambda b, h: (h, 0, 0)),
        ],
        out_specs=pl.BlockSpec((1, 1, S, DH), lambda b, h: (b, h, 0, 0)),
        out_shape=jax.ShapeDtypeStruct((B, H, S, DH), jnp.float32),
    )(h_bf, wq, wk, wv, bq, bk, bv)
    return out.transpose(0, 2, 1, 3).reshape(N, D)


def _projln_body(a_ref, w_ref, b_ref, r_ref, g_ref, bb_ref, o_ref, obf_ref):
    y = _dotT32(a_ref[...], w_ref[...]) + b_ref[...] + r_ref[...]
    y = _ln(y, g_ref[...], bb_ref[...])
    o_ref[...] = y
    obf_ref[...] = y.astype(_bf)


def _projln(ao_bf, w_bf, bias, resid, g, b):
    return pl.pallas_call(
        _projln_body,
        grid=(NT,),
        in_specs=[
            pl.BlockSpec((128, D), lambda t: (t, 0)),
            pl.BlockSpec((D, D), lambda t: (0, 0)),
            pl.BlockSpec((1, D), lambda t: (0, 0)),
            pl.BlockSpec((128, D), lambda t: (t, 0)),
            pl.BlockSpec((1, D), lambda t: (0, 0)),
            pl.BlockSpec((1, D), lambda t: (0, 0)),
        ],
        out_specs=[
            pl.BlockSpec((128, D), lambda t: (t, 0)),
            pl.BlockSpec((128, D), lambda t: (t, 0)),
        ],
        out_shape=[
            jax.ShapeDtypeStruct((N, D), jnp.float32),
            jax.ShapeDtypeStruct((N, D), _bf),
        ],
    )(ao_bf, w_bf, bias, resid, g, b)


def _resln_body(a_ref, r_ref, g_ref, bb_ref, o_ref, obf_ref):
    y = _ln(a_ref[...] + r_ref[...], g_ref[...], bb_ref[...])
    o_ref[...] = y
    obf_ref[...] = y.astype(_bf)


def _resln(a, resid, g, b):
    return pl.pallas_call(
        _resln_body,
        grid=(NT,),
        in_specs=[
            pl.BlockSpec((128, D), lambda t: (t, 0)),
            pl.BlockSpec((128, D), lambda t: (t, 0)),
            pl.BlockSpec((1, D), lambda t: (0, 0)),
            pl.BlockSpec((1, D), lambda t: (0, 0)),
        ],
        out_specs=[
            pl.BlockSpec((128, D), lambda t: (t, 0)),
            pl.BlockSpec((128, D), lambda t: (t, 0)),
        ],
        out_shape=[
            jax.ShapeDtypeStruct((N, D), jnp.float32),
            jax.ShapeDtypeStruct((N, D), _bf),
        ],
    )(a, resid, g, b)


def _gate_body(h_ref, wg_ref, bg_ref, wgt_ref, aux_ref):
    gl = _dotT32(h_ref[...], wg_ref[...]) + bg_ref[...]
    mx = jnp.max(gl, axis=-1, keepdims=True)
    pe = jnp.exp(gl - mx)
    pe = pe / jnp.sum(pe, axis=-1, keepdims=True)
    usage = jnp.mean(pe, axis=0, keepdims=True)
    aux_ref[...] = E * jnp.sum(usage * usage, axis=-1, keepdims=True)
    iot = lax.broadcasted_iota(jnp.int32, gl.shape, 1)
    m1 = jnp.max(gl, axis=-1, keepdims=True)
    i1 = jnp.min(jnp.where(gl == m1, iot, E), axis=-1, keepdims=True)
    gl2 = jnp.where(iot == i1, -1e30, gl)
    m2 = jnp.max(gl2, axis=-1, keepdims=True)
    i2 = jnp.min(jnp.where(gl2 == m2, iot, E), axis=-1, keepdims=True)
    w1 = 1.0 / (1.0 + jnp.exp(m2 - m1))
    w2 = 1.0 - w1
    wgt = jnp.where(iot == i1, w1, 0.0) + jnp.where(iot == i2, w2, 0.0)
    wgt_ref[...] = wgt


def _gate(h_bf, wg_bf, bg):
    return pl.pallas_call(
        _gate_body,
        grid=(1,),
        in_specs=[
            pl.BlockSpec((N, D), lambda i: (0, 0)),
            pl.BlockSpec((E, D), lambda i: (0, 0)),
            pl.BlockSpec((1, E), lambda i: (0, 0)),
        ],
        out_specs=[
            pl.BlockSpec((N, E), lambda i: (0, 0)),
            pl.BlockSpec((1, 1), lambda i: (0, 0)),
        ],
        out_shape=[
            jax.ShapeDtypeStruct((N, E), jnp.float32),
            jax.ShapeDtypeStruct((1, 1), jnp.float32),
        ],
    )(h_bf, wg_bf, bg)


def _moe_body(x_ref, w1_ref, b1_ref, w2_ref, b2_ref, wgt_ref, o_ref, acc_ref):
    e = pl.program_id(0)
    t = pl.program_id(1)
    sl = pl.ds(t * 128, 128)
    t1 = _dotT(x_ref[...], w1_ref[0]) + b1_ref[0]
    t1 = _gelu(t1).astype(_bf)
    y = _dotT(t1, w2_ref[0]) + b2_ref[0]
    y = y * wgt_ref[0, 0, 0][:, None]

    @pl.when(e == 0)
    def _():
        acc_ref[sl, :] = y

    @pl.when(e > 0)
    def _():
        acc_ref[sl, :] += y

    @pl.when(e == E - 1)
    def _():
        o_ref[...] = acc_ref[sl, :]


def _moe_dense(x_bf, w1s, b1s, w2s, b2s, wgtT):
    return pl.pallas_call(
        _moe_body,
        grid=(E, NT),
        in_specs=[
            pl.BlockSpec((128, D), lambda e, t: (t, 0)),
            pl.BlockSpec((1, F, D), lambda e, t: (e, 0, 0)),
            pl.BlockSpec((1, 1, F), lambda e, t: (e, 0, 0)),
            pl.BlockSpec((1, D, F), lambda e, t: (e, 0, 0)),
            pl.BlockSpec((1, 1, D), lambda e, t: (e, 0, 0)),
            pl.BlockSpec((1, 1, 1, 128), lambda e, t: (e, t, 0, 0)),
        ],
        out_specs=pl.BlockSpec((128, D), lambda e, t: (t, 0)),
        out_shape=jax.ShapeDtypeStruct((N, D), jnp.float32),
        scratch_shapes=[pltpu.VMEM((N, D), jnp.float32)],
    )(x_bf, w1s, b1s, w2s, b2s, wgtT)


def _pool_body(h_ref, hl_ref, wq_ref, wk_ref, wv_ref, bq_ref, bk_ref, bv_ref,
               wo_ref, bo_ref,
               aw1_ref, ab1_ref, ag1_ref, agb1_ref, aw2_ref, ab2_ref,
               ag2_ref, agb2_ref, aw3_ref, ab3_ref,
               pw1_ref, pb1_ref, pg1_ref, pgb1_ref, pw2_ref, pb2_ref,
               pg2_ref, pgb2_ref, pw3_ref, pb3_ref,
               act_ref, prof_ref):
    hl8 = jnp.concatenate(
        [hl_ref[...], jnp.zeros((8 - B, D), jnp.float32)], axis=0)
    q = _dotT32(hl8, wq_ref[...]) + bq_ref[...]
    rows = lax.broadcasted_iota(jnp.int32, (H, D), 0)
    cols = lax.broadcasted_iota(jnp.int32, (H, D), 1)
    mask = jnp.where(cols // DH == rows, 1.0, 0.0)
    o_rows = []
    for b in range(B):
        hb = h_ref[b]
        kb = _dotT32(hb, wk_ref[...]) + bk_ref[...]
        vb = _dotT32(hb, wv_ref[...]) + bv_ref[...]
        qp = mask * q[b:b + 1]
        sc = _dotT32(qp, kb) * (1.0 / math.sqrt(DH))
        m = jnp.max(sc, axis=-1, keepdims=True)
        p = jnp.exp(sc - m)
        p = p / jnp.sum(p, axis=-1, keepdims=True)
        o_all = _dot3(p, vb, ((1,), (0,)))
        o_rows.append(jnp.sum(o_all * mask, axis=0, keepdims=True))
    o_rows.append(jnp.zeros((8 - B, D), jnp.float32))
    o = jnp.concatenate(o_rows, axis=0)
    pooled = _dotT32(o, wo_ref[...]) + bo_ref[...]
    a1 = _ln(_gelu(_dotT32(pooled, aw1_ref[...]) + ab1_ref[...]), ag1_ref[...], agb1_ref[...])
    a2 = _ln(_gelu(_dotT32(a1, aw2_ref[...]) + ab2_ref[...]), ag2_ref[...], agb2_ref[...])
    act_ref[...] = (_dotT32(a2, aw3_ref[...]) + ab3_ref[...])[:B, :OUT]
    p1 = _gelu(_ln(_dotT32(pooled, pw1_ref[...]) + pb1_ref[...], pg1_ref[...], pgb1_ref[...]))
    p2 = _gelu(_ln(_dotT32(p1, pw2_ref[...]) + pb2_ref[...], pg2_ref[...], pgb2_ref[...]))
    prof_ref[...] = (_dotT32(p2, pw3_ref[...]) + pb3_ref[...])[:B, :1]


def _pool_heads(h_bf, hl_bf, pool_w, ap, pp):
    ins = [h_bf, hl_bf] + pool_w + ap + pp
    specs = [pl.BlockSpec(a.shape, functools.partial(lambda r, i: (0,) * r, a.ndim))
             for a in ins]
    return pl.pallas_call(
        _pool_body,
        grid=(1,),
        in_specs=specs,
        out_specs=[
            pl.BlockSpec((B, OUT), lambda i: (0, 0)),
            pl.BlockSpec((B, 1), lambda i: (0, 0)),
        ],
        out_shape=[
            jax.ShapeDtypeStruct((B, OUT), jnp.float32),
            jax.ShapeDtypeStruct((B, 1), jnp.float32),
        ],
    )(*ins)


def _pad8(a, axis=0):
    pads = [(0, 0)] * a.ndim
    pads[axis] = (0, 8 - a.shape[axis])
    return jnp.pad(a, pads)


def kernel(x, params, pos_enc):
    x2d = x.reshape(N, IN)
    pe = pos_enc[0, :S, :]

    ip = params['in_proj']
    h, h_bf = _inproj(x2d, ip['w'], ip['b'][None], pe)

    aux_total = jnp.zeros((), jnp.float32)
    for lp in params['layers']:
        at = lp['attn']
        iw = at['in_w'].astype(_bf)
        wq = iw[:D].reshape(H, DH, D)
        wk = iw[D:2 * D].reshape(H, DH, D)
        wv = iw[2 * D:].reshape(H, DH, D)
        ib = at['in_b']
        bq = ib[:D].reshape(H, 1, DH)
        bk = ib[D:2 * D].reshape(H, 1, DH)
        bv = ib[2 * D:].reshape(H, 1, DH)
        ao = _attention(h_bf.reshape(B, S, D), wq, wk, wv, bq, bk, bv)
        h, h_bf = _projln(ao, at['out']['w'], at['out']['b'][None],
                          h, lp['n1']['g'][None], lp['n1']['b'][None])

        wgt, aux = _gate(h, lp['gate']['w'], lp['gate']['b'][None])
        aux_total = aux_total + aux[0, 0]

        w1s = jnp.stack([e['l1']['w'] for e in lp['experts']]).astype(_bf)
        b1s = jnp.stack([e['l1']['b'] for e in lp['experts']])[:, None, :]
        w2s = jnp.stack([e['l2']['w'] for e in lp['experts']]).astype(_bf)
        b2s = jnp.stack([e['l2']['b'] for e in lp['experts']])[:, None, :]
        wgtT = wgt.T.reshape(E, NT, 1, 128)
        mo = _moe_dense(h_bf, w1s, b1s, w2s, b2s, wgtT)
        h, h_bf = _resln(mo, h, lp['n2']['g'][None], lp['n2']['b'][None])

    pw = params['pool']
    piw = pw['in_w']
    pool_w = [piw[:D], piw[D:2 * D], piw[2 * D:],
              pw['in_b'][None, :D], pw['in_b'][None, D:2 * D], pw['in_b'][None, 2 * D:],
              pw['out']['w'], pw['out']['b'][None]]
    apm = params['action']
    ap = [apm['l1']['w'], apm['l1']['b'][None],
          apm['n1']['g'][None], apm['n1']['b'][None],
          apm['l2']['w'], apm['l2']['b'][None],
          apm['n2']['g'][None], apm['n2']['b'][None],
          _pad8(apm['l3']['w']), _pad8(apm['l3']['b'][None], axis=1)]
    ppm = params['profit']
    pp = [ppm['l1']['w'], ppm['l1']['b'][None],
          ppm['n1']['g'][None], ppm['n1']['b'][None],
          ppm['l2']['w'], ppm['l2']['b'][None],
          ppm['n2']['g'][None], ppm['n2']['b'][None],
          _pad8(ppm['l3']['w']), _pad8(ppm['l3']['b'][None], axis=1)]

    h3 = h.reshape(B, S, D)
    hl = h3[:, S - 1, :]
    action, profit = _pool_heads(h3, hl, pool_w, ap, pp)
    return action, profit, aux_total
